# baseline (speedup 1.0000x reference)
import functools

import jax
import jax.numpy as jnp
from jax import lax
from jax.experimental import pallas as pl
from jax.experimental.pallas import tpu as pltpu
from jax.experimental.pallas import tpu_sc as plsc

_K = 5
_KK = _K * _K
_NC = 2
_NS = 16
_NW = _NC * _NS
_C = 80


def _round_up(v, m):
    return (v + m - 1) // m * m


@functools.lru_cache(maxsize=None)
def _make_edge_agg(n_out, dp, e_total, with_deg):
    npt = _round_up(-(-n_out // _NS), _C)
    n_pad = _NS * npt
    epw = e_total // _NW
    assert epw * _NW == e_total and epw % _C == 0
    nchunks = epw // _C

    out_types = [jax.ShapeDtypeStruct((_NC, n_pad, dp), jnp.float32)]
    if with_deg:
        out_types.append(jax.ShapeDtypeStruct((_NC, n_pad, 16), jnp.float32))

    scratch = [
        pltpu.VMEM((2, 4, _C), jnp.int32),
        pltpu.VMEM((2, 4, _C), jnp.float32),
        pltpu.VMEM((_C,), jnp.int32),
        pltpu.VMEM((_C,), jnp.int32),
        pltpu.VMEM((2, 4, _C, dp), jnp.bfloat16),
        pltpu.VMEM((_C, dp), jnp.float32),
        pltpu.VMEM((_C, dp), jnp.float32),
        pltpu.VMEM_SHARED((n_pad, dp), jnp.float32),
        pltpu.SemaphoreType.DMA,
        pltpu.SemaphoreType.DMA,
        pltpu.SemaphoreType.DMA,
        pltpu.SemaphoreType.DMA,
        pltpu.SemaphoreType.DMA,
        pltpu.SemaphoreType.DMA,
    ]
    if with_deg:
        scratch += [
            pltpu.VMEM((_C, 16), jnp.float32),
            pltpu.VMEM((_C, 16), jnp.float32),
            pltpu.VMEM_SHARED((n_pad, 16), jnp.float32),
        ]

    mesh = plsc.VectorSubcoreMesh(core_axis_name="c", subcore_axis_name="s",
                                  num_cores=_NC, num_subcores=_NS)

    def body(z_hbm, g_hbm, fx_hbm, fy_hbm, dst_hbm, acc_out, *rest):
        if with_deg:
            deg_out = rest[0]
            rest = rest[1:]
        (ib, wb, db0, db1, rows, abuf, zbuf, acc_sh,
         sem_i0, sem_i1, sem_w0, sem_w1, sem_g0, sem_g1) = rest[:14]
        if with_deg:
            ones, z16, deg_sh = rest[14:]
        dbs = (db0, db1)
        sems_i = (sem_i0, sem_i1)
        sems_w = (sem_w0, sem_w1)
        sems_g = (sem_g0, sem_g1)
        c = lax.axis_index("c")
        s = lax.axis_index("s")
        w = s * _NC + c

        def zrow(i, _):
            for j in range(dp // 16):
                zbuf[i, pl.ds(j * 16, 16)] = jnp.zeros((16,), jnp.float32)
            if with_deg:
                ones[i, pl.ds(0, 16)] = jnp.ones((16,), jnp.float32)
                z16[i, pl.ds(0, 16)] = jnp.zeros((16,), jnp.float32)
            return 0

        lax.fori_loop(0, _C, zrow, 0)

        def zcp(i, _):
            pltpu.sync_copy(zbuf, acc_sh.at[pl.ds(s * npt + i * _C, _C)])
            if with_deg:
                pltpu.sync_copy(z16, deg_sh.at[pl.ds(s * npt + i * _C, _C)])
            return 0

        lax.fori_loop(0, npt // _C, zcp, 0)
        plsc.subcore_barrier()

        def eload_ib(i, b):
            i = jnp.minimum(i, nchunks - 1)
            base = w * epw + i * _C
            pltpu.async_copy(g_hbm.at[pl.ds(base, _C)], ib.at[b, 0],
                             sems_i[b])

        def ewait_ib(b):
            pltpu.make_async_copy(g_hbm.at[pl.ds(0, _C)],
                                  ib.at[b, 0], sems_i[b]).wait()

        def iexpand(b):
            def ix(g, _):
                sl = pl.ds(g * 16, 16)
                v = ib[b, 0, sl]
                ib[b, 1, sl] = v + 1
                ib[b, 2, sl] = v + _K
                ib[b, 3, sl] = v + _K + 1
                return 0

            lax.fori_loop(0, _C // 16, ix, 0)

        def eload_wd(i, b):
            i = jnp.minimum(i, nchunks - 1)
            base = w * epw + i * _C
            pltpu.async_copy(fx_hbm.at[pl.ds(base, _C)], wb.at[b, 0],
                             sems_w[b])
            pltpu.async_copy(fy_hbm.at[pl.ds(base, _C)], wb.at[b, 1],
                             sems_w[b])
            pltpu.async_copy(dst_hbm.at[pl.ds(base, _C)], dbs[b], sems_w[b])

        def ewait_wd(b):
            pltpu.make_async_copy(fx_hbm.at[pl.ds(0, _C)],
                                  wb.at[b, 0], sems_w[b]).wait()
            pltpu.make_async_copy(fy_hbm.at[pl.ds(0, _C)],
                                  wb.at[b, 1], sems_w[b]).wait()
            pltpu.make_async_copy(dst_hbm.at[pl.ds(0, _C)],
                                  dbs[b], sems_w[b]).wait()

        def wexpand(b):
            def wx(g, _):
                sl = pl.ds(g * 16, 16)
                fx = wb[b, 0, sl]
                fy = wb[b, 1, sl]
                gx = 1.0 - fx
                gy = 1.0 - fy
                wb[b, 0, sl] = gx * gy
                wb[b, 1, sl] = gx * fy
                wb[b, 2, sl] = fx * gy
                wb[b, 3, sl] = fx * fy
                return 0

            lax.fori_loop(0, _C // 16, wx, 0)

        def gstart(b):
            for t in range(4):
                pltpu.async_copy(z_hbm.at[ib.at[b, t]],
                                 rows.at[b, t], sems_g[b])

        def gwait(b):
            for t in range(4):
                pltpu.make_async_copy(z_hbm.at[ib.at[b, t]],
                                      rows.at[b, t], sems_g[b]).wait()

        def process(b):
            wexpand(b)

            def cg(g, _):
                e0 = g * 16
                b0 = wb[b, 0, pl.ds(e0, 16)]
                b1 = wb[b, 1, pl.ds(e0, 16)]
                b2 = wb[b, 2, pl.ds(e0, 16)]
                b3 = wb[b, 3, pl.ds(e0, 16)]
                for lane in range(16):
                    e = e0 + lane
                    s0, s1, s2, s3 = b0[lane], b1[lane], b2[lane], b3[lane]
                    for j in range(dp // 32):
                        sl = pl.ds(j * 32, 32)
                        ev0, od0 = plsc.unpack(
                            rows[b, 0, e, sl],
                            format=plsc.PackFormat.INTERLEAVED,
                            preferred_element_type=jnp.float32)
                        ev1, od1 = plsc.unpack(
                            rows[b, 1, e, sl],
                            format=plsc.PackFormat.INTERLEAVED,
                            preferred_element_type=jnp.float32)
                        ev2, od2 = plsc.unpack(
                            rows[b, 2, e, sl],
                            format=plsc.PackFormat.INTERLEAVED,
                            preferred_element_type=jnp.float32)
                        ev3, od3 = plsc.unpack(
                            rows[b, 3, e, sl],
                            format=plsc.PackFormat.INTERLEAVED,
                            preferred_element_type=jnp.float32)
                        abuf[e, pl.ds(j * 32, 16)] = (
                            ev0 * s0 + ev1 * s1 + ev2 * s2 + ev3 * s3)
                        abuf[e, pl.ds(j * 32 + 16, 16)] = (
                            od0 * s0 + od1 * s1 + od2 * s2 + od3 * s3)
                return 0

            lax.fori_loop(0, _C // 16, cg, 0)
            pltpu.sync_copy(abuf, acc_sh.at[dbs[b]], add=True)
            if with_deg:
                pltpu.sync_copy(ones, deg_sh.at[dbs[b]], add=True)

        eload_ib(0, 0)
        eload_wd(0, 0)
        ewait_ib(0)
        iexpand(0)
        gstart(0)
        eload_ib(1, 1)
        eload_wd(1, 1)
        ewait_ib(1)
        iexpand(1)

        def pair_body(j, _):
            i = j * 2
            gstart(1)
            gwait(0)
            eload_ib(i + 2, 0)
            ewait_wd(0)
            process(0)
            eload_wd(i + 2, 0)
            ewait_ib(0)
            iexpand(0)
            gstart(0)
            gwait(1)
            eload_ib(i + 3, 1)
            ewait_wd(1)
            process(1)
            eload_wd(i + 3, 1)
            ewait_ib(1)
            iexpand(1)
            return 0

        lax.fori_loop(0, nchunks // 2, pair_body, 0)
        gwait(0)
        ewait_wd(0)
        ewait_wd(1)
        plsc.subcore_barrier()

        pltpu.sync_copy(acc_sh.at[pl.ds(s * npt, npt)],
                        acc_out.at[c, pl.ds(s * npt, npt)])
        if with_deg:
            pltpu.sync_copy(deg_sh.at[pl.ds(s * npt, npt)],
                            deg_out.at[c, pl.ds(s * npt, npt)])

    kern = pl.kernel(
        body,
        out_type=tuple(out_types) if with_deg else out_types[0],
        mesh=mesh,
        scratch_types=scratch,
        compiler_params=pltpu.CompilerParams(use_tc_tiling_on_sc=False, needs_layout_passes=False),
    )
    return kern, n_pad


@functools.lru_cache(maxsize=None)
def _make_cart(n_nodes, e_total):
    epw = e_total // _NW
    c2 = 400
    assert epw % c2 == 0
    nchunks = epw // c2

    out_types = (
        jax.ShapeDtypeStruct((2 * e_total,), jnp.float32),
        jax.ShapeDtypeStruct((_NW * 16,), jnp.float32),
    )
    scratch = [
        pltpu.VMEM((2 * n_nodes,), jnp.float32),
        pltpu.VMEM((c2,), jnp.int32),
        pltpu.VMEM((c2,), jnp.int32),
        pltpu.VMEM((c2,), jnp.float32),
        pltpu.VMEM((c2,), jnp.float32),
        pltpu.VMEM((16,), jnp.float32),
    ]
    mesh = plsc.VectorSubcoreMesh(core_axis_name="c", subcore_axis_name="s",
                                  num_cores=_NC, num_subcores=_NS)

    def body(pos_hbm, src_hbm, dst_hbm, cart_out, max_out,
             pos_v, sbuf, dbuf, cxbuf, cybuf, mbuf):
        c = lax.axis_index("c")
        s = lax.axis_index("s")
        w = s * _NC + c
        pltpu.sync_copy(pos_hbm, pos_v)

        def chunk(i, m):
            base = w * epw + i * c2
            pltpu.sync_copy(src_hbm.at[pl.ds(base, c2)], sbuf)
            pltpu.sync_copy(dst_hbm.at[pl.ds(base, c2)], dbuf)

            def grp(g, m):
                s16 = sbuf[pl.ds(g * 16, 16)] * 2
                d16 = dbuf[pl.ds(g * 16, 16)] * 2
                one = jnp.ones((16,), jnp.int32)
                cx = plsc.load_gather(pos_v, [s16]) - plsc.load_gather(pos_v, [d16])
                cy = (plsc.load_gather(pos_v, [s16 + one])
                      - plsc.load_gather(pos_v, [d16 + one]))
                cxbuf[pl.ds(g * 16, 16)] = cx
                cybuf[pl.ds(g * 16, 16)] = cy
                return jnp.maximum(m, jnp.maximum(jnp.abs(cx), jnp.abs(cy)))

            m = lax.fori_loop(0, c2 // 16, grp, m)
            pltpu.sync_copy(cxbuf, cart_out.at[pl.ds(base, c2)])
            pltpu.sync_copy(cybuf, cart_out.at[pl.ds(e_total + base, c2)])
            return m

        m = lax.fori_loop(0, nchunks, chunk, jnp.zeros((16,), jnp.float32))
        mbuf[pl.ds(0, 16)] = m
        pltpu.sync_copy(mbuf, max_out.at[pl.ds(w * 16, 16)])

    return pl.kernel(
        body,
        out_type=out_types,
        mesh=mesh,
        scratch_types=scratch,
        compiler_params=pltpu.CompilerParams(use_tc_tiling_on_sc=False, needs_layout_passes=False),
    )


def _cart_sc(pos, src, dst):
    n = pos.shape[0]
    E = src.shape[0]
    cart_flat, maxes = _make_cart(n, E)(pos.reshape(-1), src, dst)
    m = jnp.max(maxes)
    cart = jnp.stack([cart_flat[:E], cart_flat[E:]], axis=1)
    return cart / (2.0 * m) + 0.5


def _edge_prep(pseudo, src):
    scaled = jnp.clip(pseudo, 0.0, 1.0) * (_K - 1)
    i0 = jnp.clip(jnp.floor(scaled).astype(jnp.int32), 0, _K - 2)
    frac = scaled - i0.astype(scaled.dtype)
    g0 = src * _KK + i0[:, 0] * _K + i0[:, 1]
    return g0, frac[:, 0], frac[:, 1]


def _conv_sc(x, dst, g0, fx, fy, W, root, bias, num_nodes, deg):
    dout = W.shape[2]
    dp = max(32, _round_up(dout, 32))
    if dp != dout:
        W = jnp.pad(W, ((0, 0), (0, 0), (0, dp - dout)))
    q = jnp.arange(dp)
    W = W[:, :, (q // 32) * 32 + (q % 32) // 2 + (q % 2) * 16]
    Z = (jnp.einsum("nd,kdo->nko", x, W)
         .astype(jnp.bfloat16).reshape(num_nodes * _KK, dp))
    E = dst.shape[0]
    with_deg = deg is None
    kern, n_pad = _make_edge_agg(num_nodes, dp, E, with_deg)
    args = (Z, g0, fx, fy, dst)
    if with_deg:
        acc2, deg2 = kern(*args)
        deg = (deg2[0] + deg2[1])[:num_nodes, 0]
    else:
        acc2 = kern(*args)
    acc = (acc2[0] + acc2[1])[:num_nodes, :dout]
    out = acc / jnp.clip(deg, 1.0)[:, None]
    return out + x @ root + bias, deg


def _cart(pos, src, dst):
    cart = pos[src] - pos[dst]
    m = jnp.max(jnp.abs(cart))
    return cart / (2.0 * m) + 0.5


def kernel(x, edge_index, edge_attr, pos, W1, root1, b1, W2, root2, b2,
           W3, root3, b3, W4, root4, b4):
    src = edge_index[0].astype(jnp.int32)
    dst = edge_index[1].astype(jnp.int32)
    N = x.shape[0]
    N1 = N // 2
    N2 = N1 // 2

    src1 = src // 2
    dst1 = dst // 2

    g0a, fxa, fya = _edge_prep(edge_attr, src)
    h, deg0 = _conv_sc(x, dst, g0a, fxa, fya, W1, root1, b1, N, None)
    h = jax.nn.elu(h)
    deg1 = deg0.reshape(N1, 2).sum(axis=1)

    h1 = h.reshape(N1, 2, -1).max(axis=1)
    pos1 = pos.reshape(N1, 2, 2).mean(axis=1)

    ea1 = _cart_sc(pos1, src1, dst1)
    g0b, fxb, fyb = _edge_prep(ea1, src1)
    h1, _ = _conv_sc(h1, dst1, g0b, fxb, fyb, W2, root2, b2, N1, deg1)
    h1 = jax.nn.elu(h1)

    h2 = h1.reshape(N2, 2, -1).max(axis=1)
    hr1 = jnp.repeat(h2, 2, axis=0)

    hr1, _ = _conv_sc(hr1, dst1, g0b, fxb, fyb, W3, root3, b3, N1, deg1)
    hr1 = jax.nn.elu(hr1)

    hr0 = jnp.repeat(hr1, 2, axis=0)

    ea0 = _cart_sc(pos, src, dst)
    g0c, fxc, fyc = _edge_prep(ea0, src)
    hr0, _ = _conv_sc(hr0, dst, g0c, fxc, fyc, W4, root4, b4, N, deg0)
    hr0 = jax.nn.elu(hr0)
    return jax.nn.sigmoid(hr0)


# speedup vs baseline: 1.0875x; 1.0875x over previous
import functools

import jax
import jax.numpy as jnp
from jax import lax
from jax.experimental import pallas as pl
from jax.experimental.pallas import tpu as pltpu
from jax.experimental.pallas import tpu_sc as plsc

_K = 5
_KK = _K * _K
_NC = 2
_NS = 16
_NW = _NC * _NS
_C = 80


def _round_up(v, m):
    return (v + m - 1) // m * m


@functools.lru_cache(maxsize=None)
def _make_edge_agg(n_out, dp, e_total, with_deg):
    npt = _round_up(-(-n_out // _NS), _C)
    n_pad = _NS * npt
    epw = e_total // _NW
    assert epw * _NW == e_total and epw % _C == 0
    nchunks = epw // _C

    out_types = [jax.ShapeDtypeStruct((_NC, n_pad, dp), jnp.float32)]
    if with_deg:
        out_types.append(jax.ShapeDtypeStruct((_NC, n_pad, 16), jnp.float32))

    scratch = [
        pltpu.VMEM((2, 4, _C), jnp.int32),
        pltpu.VMEM((2, 4, _C), jnp.float32),
        pltpu.VMEM((_C,), jnp.int32),
        pltpu.VMEM((_C,), jnp.int32),
        pltpu.VMEM((2, 4, _C, dp), jnp.float32),
        pltpu.VMEM((_C, dp), jnp.float32),
        pltpu.VMEM((_C, dp), jnp.float32),
        pltpu.VMEM_SHARED((n_pad, dp), jnp.float32),
        pltpu.SemaphoreType.DMA,
        pltpu.SemaphoreType.DMA,
        pltpu.SemaphoreType.DMA,
        pltpu.SemaphoreType.DMA,
        pltpu.SemaphoreType.DMA,
        pltpu.SemaphoreType.DMA,
    ]
    if with_deg:
        scratch += [
            pltpu.VMEM((_C, 16), jnp.float32),
            pltpu.VMEM((_C, 16), jnp.float32),
            pltpu.VMEM_SHARED((n_pad, 16), jnp.float32),
        ]

    mesh = plsc.VectorSubcoreMesh(core_axis_name="c", subcore_axis_name="s",
                                  num_cores=_NC, num_subcores=_NS)

    def body(z_hbm, g_hbm, fx_hbm, fy_hbm, dst_hbm, acc_out, *rest):
        if with_deg:
            deg_out = rest[0]
            rest = rest[1:]
        (ib, wb, db0, db1, rows, abuf, zbuf, acc_sh,
         sem_i0, sem_i1, sem_w0, sem_w1, sem_g0, sem_g1) = rest[:14]
        if with_deg:
            ones, z16, deg_sh = rest[14:]
        dbs = (db0, db1)
        sems_i = (sem_i0, sem_i1)
        sems_w = (sem_w0, sem_w1)
        sems_g = (sem_g0, sem_g1)
        c = lax.axis_index("c")
        s = lax.axis_index("s")
        w = s * _NC + c

        def zrow(i, _):
            for j in range(dp // 16):
                zbuf[i, pl.ds(j * 16, 16)] = jnp.zeros((16,), jnp.float32)
            if with_deg:
                ones[i, pl.ds(0, 16)] = jnp.ones((16,), jnp.float32)
                z16[i, pl.ds(0, 16)] = jnp.zeros((16,), jnp.float32)
            return 0

        lax.fori_loop(0, _C, zrow, 0)

        def zcp(i, _):
            pltpu.sync_copy(zbuf, acc_sh.at[pl.ds(s * npt + i * _C, _C)])
            if with_deg:
                pltpu.sync_copy(z16, deg_sh.at[pl.ds(s * npt + i * _C, _C)])
            return 0

        lax.fori_loop(0, npt // _C, zcp, 0)
        plsc.subcore_barrier()

        def eload_ib(i, b):
            i = jnp.minimum(i, nchunks - 1)
            base = w * epw + i * _C
            pltpu.async_copy(g_hbm.at[pl.ds(base, _C)], ib.at[b, 0],
                             sems_i[b])

        def ewait_ib(b):
            pltpu.make_async_copy(g_hbm.at[pl.ds(0, _C)],
                                  ib.at[b, 0], sems_i[b]).wait()

        def iexpand(b):
            def ix(g, _):
                sl = pl.ds(g * 16, 16)
                v = ib[b, 0, sl]
                ib[b, 1, sl] = v + 1
                ib[b, 2, sl] = v + _K
                ib[b, 3, sl] = v + _K + 1
                return 0

            lax.fori_loop(0, _C // 16, ix, 0)

        def eload_wd(i, b):
            i = jnp.minimum(i, nchunks - 1)
            base = w * epw + i * _C
            pltpu.async_copy(fx_hbm.at[pl.ds(base, _C)], wb.at[b, 0],
                             sems_w[b])
            pltpu.async_copy(fy_hbm.at[pl.ds(base, _C)], wb.at[b, 1],
                             sems_w[b])
            pltpu.async_copy(dst_hbm.at[pl.ds(base, _C)], dbs[b], sems_w[b])

        def ewait_wd(b):
            pltpu.make_async_copy(fx_hbm.at[pl.ds(0, _C)],
                                  wb.at[b, 0], sems_w[b]).wait()
            pltpu.make_async_copy(fy_hbm.at[pl.ds(0, _C)],
                                  wb.at[b, 1], sems_w[b]).wait()
            pltpu.make_async_copy(dst_hbm.at[pl.ds(0, _C)],
                                  dbs[b], sems_w[b]).wait()

        def wexpand(b):
            def wx(g, _):
                sl = pl.ds(g * 16, 16)
                fx = wb[b, 0, sl]
                fy = wb[b, 1, sl]
                gx = 1.0 - fx
                gy = 1.0 - fy
                wb[b, 0, sl] = gx * gy
                wb[b, 1, sl] = gx * fy
                wb[b, 2, sl] = fx * gy
                wb[b, 3, sl] = fx * fy
                return 0

            lax.fori_loop(0, _C // 16, wx, 0)

        def gstart(b):
            for t in range(4):
                pltpu.async_copy(z_hbm.at[ib.at[b, t]],
                                 rows.at[b, t], sems_g[b])

        def gwait(b):
            for t in range(4):
                pltpu.make_async_copy(z_hbm.at[ib.at[b, t]],
                                      rows.at[b, t], sems_g[b]).wait()

        def process(b):
            wexpand(b)

            def cg(g, _):
                e0 = g * 16
                b0 = wb[b, 0, pl.ds(e0, 16)]
                b1 = wb[b, 1, pl.ds(e0, 16)]
                b2 = wb[b, 2, pl.ds(e0, 16)]
                b3 = wb[b, 3, pl.ds(e0, 16)]
                for lane in range(16):
                    e = e0 + lane
                    s0, s1, s2, s3 = b0[lane], b1[lane], b2[lane], b3[lane]
                    for j in range(dp // 16):
                        sl = pl.ds(j * 16, 16)
                        abuf[e, sl] = (rows[b, 0, e, sl] * s0
                                       + rows[b, 1, e, sl] * s1
                                       + rows[b, 2, e, sl] * s2
                                       + rows[b, 3, e, sl] * s3)
                return 0

            lax.fori_loop(0, _C // 16, cg, 0)
            pltpu.sync_copy(abuf, acc_sh.at[dbs[b]], add=True)
            if with_deg:
                pltpu.sync_copy(ones, deg_sh.at[dbs[b]], add=True)

        eload_ib(0, 0)
        eload_wd(0, 0)
        ewait_ib(0)
        iexpand(0)
        gstart(0)
        eload_ib(1, 1)
        eload_wd(1, 1)
        ewait_ib(1)
        iexpand(1)

        def pair_body(j, _):
            i = j * 2
            gstart(1)
            gwait(0)
            eload_ib(i + 2, 0)
            ewait_wd(0)
            process(0)
            eload_wd(i + 2, 0)
            ewait_ib(0)
            iexpand(0)
            gstart(0)
            gwait(1)
            eload_ib(i + 3, 1)
            ewait_wd(1)
            process(1)
            eload_wd(i + 3, 1)
            ewait_ib(1)
            iexpand(1)
            return 0

        lax.fori_loop(0, nchunks // 2, pair_body, 0)
        gwait(0)
        ewait_wd(0)
        ewait_wd(1)
        plsc.subcore_barrier()

        pltpu.sync_copy(acc_sh.at[pl.ds(s * npt, npt)],
                        acc_out.at[c, pl.ds(s * npt, npt)])
        if with_deg:
            pltpu.sync_copy(deg_sh.at[pl.ds(s * npt, npt)],
                            deg_out.at[c, pl.ds(s * npt, npt)])

    kern = pl.kernel(
        body,
        out_type=tuple(out_types) if with_deg else out_types[0],
        mesh=mesh,
        scratch_types=scratch,
        compiler_params=pltpu.CompilerParams(use_tc_tiling_on_sc=False, needs_layout_passes=False),
    )
    return kern, n_pad


@functools.lru_cache(maxsize=None)
def _make_cart(n_nodes, e_total):
    epw = e_total // _NW
    c2 = 400
    assert epw % c2 == 0
    nchunks = epw // c2

    out_types = (
        jax.ShapeDtypeStruct((2 * e_total,), jnp.float32),
        jax.ShapeDtypeStruct((_NW * 16,), jnp.float32),
    )
    scratch = [
        pltpu.VMEM((2 * n_nodes,), jnp.float32),
        pltpu.VMEM((c2,), jnp.int32),
        pltpu.VMEM((c2,), jnp.int32),
        pltpu.VMEM((c2,), jnp.float32),
        pltpu.VMEM((c2,), jnp.float32),
        pltpu.VMEM((16,), jnp.float32),
    ]
    mesh = plsc.VectorSubcoreMesh(core_axis_name="c", subcore_axis_name="s",
                                  num_cores=_NC, num_subcores=_NS)

    def body(pos_hbm, src_hbm, dst_hbm, cart_out, max_out,
             pos_v, sbuf, dbuf, cxbuf, cybuf, mbuf):
        c = lax.axis_index("c")
        s = lax.axis_index("s")
        w = s * _NC + c
        pltpu.sync_copy(pos_hbm, pos_v)

        def chunk(i, m):
            base = w * epw + i * c2
            pltpu.sync_copy(src_hbm.at[pl.ds(base, c2)], sbuf)
            pltpu.sync_copy(dst_hbm.at[pl.ds(base, c2)], dbuf)

            def grp(g, m):
                s16 = sbuf[pl.ds(g * 16, 16)] * 2
                d16 = dbuf[pl.ds(g * 16, 16)] * 2
                one = jnp.ones((16,), jnp.int32)
                cx = plsc.load_gather(pos_v, [s16]) - plsc.load_gather(pos_v, [d16])
                cy = (plsc.load_gather(pos_v, [s16 + one])
                      - plsc.load_gather(pos_v, [d16 + one]))
                cxbuf[pl.ds(g * 16, 16)] = cx
                cybuf[pl.ds(g * 16, 16)] = cy
                return jnp.maximum(m, jnp.maximum(jnp.abs(cx), jnp.abs(cy)))

            m = lax.fori_loop(0, c2 // 16, grp, m)
            pltpu.sync_copy(cxbuf, cart_out.at[pl.ds(base, c2)])
            pltpu.sync_copy(cybuf, cart_out.at[pl.ds(e_total + base, c2)])
            return m

        m = lax.fori_loop(0, nchunks, chunk, jnp.zeros((16,), jnp.float32))
        mbuf[pl.ds(0, 16)] = m
        pltpu.sync_copy(mbuf, max_out.at[pl.ds(w * 16, 16)])

    return pl.kernel(
        body,
        out_type=out_types,
        mesh=mesh,
        scratch_types=scratch,
        compiler_params=pltpu.CompilerParams(use_tc_tiling_on_sc=False, needs_layout_passes=False),
    )


def _cart_sc(pos, src, dst):
    n = pos.shape[0]
    E = src.shape[0]
    cart_flat, maxes = _make_cart(n, E)(pos.reshape(-1), src, dst)
    m = jnp.max(maxes)
    cart = jnp.stack([cart_flat[:E], cart_flat[E:]], axis=1)
    return cart / (2.0 * m) + 0.5


def _edge_prep(pseudo, src):
    scaled = jnp.clip(pseudo, 0.0, 1.0) * (_K - 1)
    i0 = jnp.clip(jnp.floor(scaled).astype(jnp.int32), 0, _K - 2)
    frac = scaled - i0.astype(scaled.dtype)
    g0 = src * _KK + i0[:, 0] * _K + i0[:, 1]
    return g0, frac[:, 0], frac[:, 1]


def _conv_sc(x, dst, g0, fx, fy, W, root, bias, num_nodes, deg):
    dout = W.shape[2]
    dp = max(16, _round_up(dout, 16))
    if dp != dout:
        W = jnp.pad(W, ((0, 0), (0, 0), (0, dp - dout)))
    Z = jnp.einsum("nd,kdo->nko", x, W).reshape(num_nodes * _KK, dp)
    E = dst.shape[0]
    with_deg = deg is None
    kern, n_pad = _make_edge_agg(num_nodes, dp, E, with_deg)
    args = (Z, g0, fx, fy, dst)
    if with_deg:
        acc2, deg2 = kern(*args)
        deg = (deg2[0] + deg2[1])[:num_nodes, 0]
    else:
        acc2 = kern(*args)
    acc = (acc2[0] + acc2[1])[:num_nodes, :dout]
    out = acc / jnp.clip(deg, 1.0)[:, None]
    return out + x @ root + bias, deg


def _cart(pos, src, dst):
    cart = pos[src] - pos[dst]
    m = jnp.max(jnp.abs(cart))
    return cart / (2.0 * m) + 0.5


def kernel(x, edge_index, edge_attr, pos, W1, root1, b1, W2, root2, b2,
           W3, root3, b3, W4, root4, b4):
    src = edge_index[0].astype(jnp.int32)
    dst = edge_index[1].astype(jnp.int32)
    N = x.shape[0]
    N1 = N // 2
    N2 = N1 // 2

    src1 = src // 2
    dst1 = dst // 2

    g0a, fxa, fya = _edge_prep(edge_attr, src)
    h, deg0 = _conv_sc(x, dst, g0a, fxa, fya, W1, root1, b1, N, None)
    h = jax.nn.elu(h)
    deg1 = deg0.reshape(N1, 2).sum(axis=1)

    h1 = h.reshape(N1, 2, -1).max(axis=1)
    pos1 = pos.reshape(N1, 2, 2).mean(axis=1)

    ea1 = _cart_sc(pos1, src1, dst1)
    g0b, fxb, fyb = _edge_prep(ea1, src1)
    h1, _ = _conv_sc(h1, dst1, g0b, fxb, fyb, W2, root2, b2, N1, deg1)
    h1 = jax.nn.elu(h1)

    h2 = h1.reshape(N2, 2, -1).max(axis=1)
    hr1 = jnp.repeat(h2, 2, axis=0)

    hr1, _ = _conv_sc(hr1, dst1, g0b, fxb, fyb, W3, root3, b3, N1, deg1)
    hr1 = jax.nn.elu(hr1)

    hr0 = jnp.repeat(hr1, 2, axis=0)

    ea0 = _cart_sc(pos, src, dst)
    g0c, fxc, fyc = _edge_prep(ea0, src)
    hr0, _ = _conv_sc(hr0, dst, g0c, fxc, fyc, W4, root4, b4, N, deg0)
    hr0 = jax.nn.elu(hr0)
    return jax.nn.sigmoid(hr0)
